# Initial kernel scaffold; baseline (speedup 1.0000x reference)
import jax
import jax.numpy as jnp
from jax.experimental import pallas as pl


def kernel(x, edge_index, W1, b1, W2, b2, Wq, bq, Wk, bk, Wv, bv, Wo, bo, Wm1, bm1, Wm2, bm2, g2, beta2, Wl, bl):
    raise NotImplementedError("write your pallas kernel here")


# speedup vs baseline: 22.8035x; 22.8035x over previous
import functools

import jax
import jax.numpy as jnp
from jax import lax
from jax.experimental import pallas as pl
from jax.experimental.pallas import tpu as pltpu
from jax.experimental.pallas import tpu_sc as plsc

T = 16
NSC = 2
BLK = 128
NPAD = 240
NBUF = 3


def _sc_mesh():
    return plsc.VectorSubcoreMesh(core_axis_name="c", subcore_axis_name="s")


def _deg_body(G, NBLK, RPT,
              dst_hbm, zeros_hbm, ones_hbm, deg_hbm,
              idx_v, ones_v, zero_v, stage_v, acc_sh):
    c = lax.axis_index("c")
    s = lax.axis_index("s")
    pltpu.sync_copy(ones_hbm, ones_v)
    pltpu.sync_copy(zeros_hbm, zero_v)
    for k in range(G // NSC):
        gi = c * (G // NSC) + k
        pltpu.sync_copy(zero_v, acc_sh.at[pl.ds(s * RPT, RPT)])
        plsc.subcore_barrier()
        pltpu.sync_copy(dst_hbm.at[gi, s], idx_v)

        def blk(b):
            pltpu.sync_copy(ones_v, acc_sh.at[idx_v.at[b]], add=True)

        pl.loop(0, NBLK)(blk)
        plsc.subcore_barrier()
        pltpu.sync_copy(acc_sh.at[pl.ds(s * RPT, RPT)], stage_v)
        pltpu.sync_copy(stage_v, deg_hbm.at[gi, pl.ds(s * RPT, RPT)])
        plsc.subcore_barrier()


def _make_deg_kernel(G, NP, NBLK):
    RPT = NP // T
    return functools.partial(
        pl.kernel,
        out_type=jax.ShapeDtypeStruct((G, NP, 16), jnp.float32),
        mesh=_sc_mesh(),
        compiler_params=pltpu.CompilerParams(use_tc_tiling_on_sc=False),
        scratch_types=[
            pltpu.VMEM((NBLK, BLK), jnp.int32),
            pltpu.VMEM((BLK, 16), jnp.float32),
            pltpu.VMEM((RPT, 16), jnp.float32),
            pltpu.VMEM((RPT, 16), jnp.float32),
            pltpu.VMEM_SHARED((NP, 16), jnp.float32),
        ],
    )(functools.partial(_deg_body, G, NBLK, RPT))


def _seg_body(G, NBLK, RPT, IBLK,
              hd_hbm, src_hbm, dst_hbm, out_hbm,
              src_v, dst_v, b0, b1, s0, s1, acc_sh):
    c = lax.axis_index("c")
    s = lax.axis_index("s")
    ring = (b0, b1)
    sems = (s0, s1)
    NCH = RPT // BLK
    NICH = NBLK // IBLK

    for k in range(G // NSC):
        gi = c * (G // NSC) + k
        for j in range(NCH):
            r0 = s * RPT + j * BLK
            pltpu.sync_copy(hd_hbm.at[gi, pl.ds(r0, BLK)], b0)
            pltpu.sync_copy(b0, acc_sh.at[pl.ds(r0, BLK)])
        plsc.subcore_barrier()

        def gather(j):
            pltpu.async_copy(hd_hbm.at[gi].at[src_v.at[j]], ring[j % 2],
                             sems[j % 2])

        def gwait(j):
            pltpu.make_async_copy(hd_hbm.at[gi].at[src_v.at[j]], ring[j % 2],
                                  sems[j % 2]).wait()

        def chunk(ic):
            pltpu.sync_copy(src_hbm.at[gi, s, pl.ds(ic * IBLK, IBLK)], src_v)
            pltpu.sync_copy(dst_hbm.at[gi, s, pl.ds(ic * IBLK, IBLK)], dst_v)
            gather(0)
            for j in range(IBLK):
                gwait(j)
                if j + 1 < IBLK:
                    gather(j + 1)
                pltpu.sync_copy(ring[j % 2], acc_sh.at[dst_v.at[j]], add=True)

        pl.loop(0, NICH)(chunk)
        plsc.subcore_barrier()
        for j in range(NCH):
            r0 = s * RPT + j * BLK
            pltpu.sync_copy(acc_sh.at[pl.ds(r0, BLK)], b1)
            pltpu.sync_copy(b1, out_hbm.at[gi, pl.ds(r0, BLK)])
        plsc.subcore_barrier()


def _make_seg_kernel(G, NP, D, NBLK, IBLK):
    RPT = NP // T
    return functools.partial(
        pl.kernel,
        out_type=jax.ShapeDtypeStruct((G, NP, D), jnp.float32),
        mesh=_sc_mesh(),
        scratch_types=[
            pltpu.VMEM((IBLK, BLK), jnp.int32),
            pltpu.VMEM((IBLK, BLK), jnp.int32),
            pltpu.VMEM((BLK, D), jnp.float32),
            pltpu.VMEM((BLK, D), jnp.float32),
            pltpu.SemaphoreType.DMA,
            pltpu.SemaphoreType.DMA,
            pltpu.VMEM_SHARED((NP, D), jnp.float32),
        ],
    )(functools.partial(_seg_body, G, NBLK, RPT, IBLK))


def _mm1_body(x_ref, deg_ref, w_ref, o_ref):
    dis = lax.rsqrt(deg_ref[0, :, 0:1] + 1.0)
    h = lax.dot_general(x_ref[0], w_ref[...], (((1,), (1,)), ((), ())),
                        preferred_element_type=jnp.float32)
    o_ref[0] = h * dis


def _mm2_body(s_ref, deg_ref, w_ref, b_ref, o_ref):
    dis = lax.rsqrt(deg_ref[0, :, 0:1] + 1.0)
    h = jnp.tanh(s_ref[0] * dis + b_ref[...])
    h = lax.dot_general(h, w_ref[...], (((1,), (1,)), ((), ())),
                        preferred_element_type=jnp.float32)
    o_ref[0] = h * dis


def _c11(a, b):
    return lax.dot_general(a, b, (((1,), (1,)), ((), ())),
                           preferred_element_type=jnp.float32)


def _head_body(G, NT, HEADS, N, R, s_ref, deg_ref, b2_ref,
               wq_ref, bq_ref, wk_ref, bk_ref, wv_ref, bv_ref,
               wo_ref, bo_ref, wm1_ref, bm1_ref, wm2_ref, bm2_ref,
               g2_ref, beta2_ref, wl_ref, bl_ref,
               lg_ref, nr_ref, xacc):
    gi = pl.program_id(0)
    nt = pl.program_id(1)
    dis = lax.rsqrt(deg_ref[0, :, 0:1] + 1.0)
    h = jnp.tanh(s_ref[0] * dis + b2_ref[...])
    rows = lax.broadcasted_iota(jnp.int32, (R, 1), 0) + nt * R
    h = jnp.where(rows < N, h, 0.0)
    part = jnp.sum(h, axis=0, keepdims=True)

    @pl.when(nt == 0)
    def _():
        xacc[pl.ds(gi, 1), :] = part

    @pl.when(nt != 0)
    def _():
        xacc[pl.ds(gi, 1), :] = xacc[pl.ds(gi, 1), :] + part

    @pl.when((gi == G - 1) & (nt == NT - 1))
    def _():
        X = xacc[0:G, :]
        q = _c11(X, wq_ref[...]) + bq_ref[...]
        k_ = _c11(X, wk_ref[...]) + bk_ref[...]
        v = _c11(X, wv_ref[...]) + bv_ref[...]
        dh = X.shape[1] // HEADS
        scale = 1.0 / (float(dh) ** 0.5)
        outs = []
        for hh in range(HEADS):
            sl = slice(hh * dh, (hh + 1) * dh)
            sc = _c11(q[:, sl], k_[:, sl]) * scale
            sc = sc - jnp.max(sc, axis=-1, keepdims=True)
            e = jnp.exp(sc)
            a = e / jnp.sum(e, axis=-1, keepdims=True)
            outs.append(lax.dot_general(a, v[:, sl], (((1,), (0,)), ((), ())),
                                        preferred_element_type=jnp.float32))
        o = jnp.concatenate(outs, axis=1)
        xa = _c11(o, wo_ref[...]) + bo_ref[...]
        m = jnp.maximum(_c11(xa, wm1_ref[...]) + bm1_ref[...], 0.0)
        m = _c11(m, wm2_ref[...]) + bm2_ref[...]
        y = xa + m
        mu = jnp.mean(y, axis=-1, keepdims=True)
        var = jnp.mean((y - mu) ** 2, axis=-1, keepdims=True)
        y = (y - mu) * lax.rsqrt(var + 1e-5) * g2_ref[...] + beta2_ref[...]
        xr = jnp.maximum(y, 0.0)
        nr = jnp.sum(xr, axis=0, keepdims=True)
        nr_ref[...] = nr
        lg_ref[...] = _c11(nr, wl_ref[...]) + bl_ref[...]


def kernel(x, edge_index, W1, b1, W2, b2, Wq, bq, Wk, bk, Wv, bv, Wo, bo,
           Wm1, bm1, Wm2, bm2, g2, beta2, Wl, bl):
    G, N, D = x.shape
    E = edge_index.shape[2]
    HID = Wm1.shape[0]
    NC = Wl.shape[0]
    HEADS = 8
    NP = N + NPAD
    EPT = E // T
    IBLK = 16
    NBLK = -(-EPT // (BLK * IBLK)) * IBLK
    PAD = NBLK * BLK - EPT
    R = 1024
    NT = NP // R

    ei = edge_index.reshape(G, 2, T, EPT)
    ar = jnp.arange(PAD, dtype=jnp.int32)
    toff = jnp.arange(T, dtype=jnp.int32)[None, :, None] * 13
    sent_src = jnp.broadcast_to((ar[None, None, :] * 37 + toff) % N,
                                (G, T, PAD))
    sent_dst = N + (ar[None, None, :] * 7 + toff) % NPAD
    sent_dst = jnp.broadcast_to(sent_dst, (G, T, PAD))
    src_p = jnp.concatenate([ei[:, 0], sent_src], axis=2).reshape(
        G, T, NBLK, BLK)
    dst_p = jnp.concatenate([ei[:, 1], sent_dst], axis=2).reshape(
        G, T, NBLK, BLK)

    zeros_c = jnp.zeros((NP // T, 16), jnp.float32)
    ones_c = jnp.ones((BLK, 16), jnp.float32)

    deg = _make_deg_kernel(G, NP, NBLK)(dst_p, zeros_c, ones_c)

    xp = jnp.pad(x, ((0, 0), (0, NPAD), (0, 0)))
    seg = _make_seg_kernel(G, NP, D, NBLK, IBLK)

    mm_grid = (G, NT)
    bs_row = pl.BlockSpec((1, R, D), lambda g, t: (g, t, 0))
    bs_deg = pl.BlockSpec((1, R, 16), lambda g, t: (g, t, 0))
    bs_w = pl.BlockSpec((D, D), lambda g, t: (0, 0))
    bs_b = pl.BlockSpec((1, D), lambda g, t: (0, 0))
    hd1 = pl.pallas_call(
        _mm1_body,
        grid=mm_grid,
        in_specs=[bs_row, bs_deg, bs_w],
        out_specs=bs_row,
        out_shape=jax.ShapeDtypeStruct((G, NP, D), jnp.float32),
    )(xp, deg, W1)
    s1 = seg(hd1, src_p, dst_p)

    hd2 = pl.pallas_call(
        _mm2_body,
        grid=mm_grid,
        in_specs=[bs_row, bs_deg, bs_w, bs_b],
        out_specs=bs_row,
        out_shape=jax.ShapeDtypeStruct((G, NP, D), jnp.float32),
    )(s1, deg, W2, b1.reshape(1, D))
    s2 = seg(hd2, src_p, dst_p)

    Wl_p = jnp.zeros((D, D), jnp.float32).at[:NC].set(Wl)
    bl_p = jnp.zeros((1, D), jnp.float32).at[0, :NC].set(bl)
    bs_hid = pl.BlockSpec((HID, D), lambda g, t: (0, 0))
    bs_bhid = pl.BlockSpec((1, HID), lambda g, t: (0, 0))
    bs_wm2 = pl.BlockSpec((D, HID), lambda g, t: (0, 0))
    bs_out = pl.BlockSpec((1, D), lambda g, t: (0, 0))
    lg, nr = pl.pallas_call(
        functools.partial(_head_body, G, NT, HEADS, N, R),
        grid=mm_grid,
        in_specs=[bs_row, bs_deg, bs_b,
                  bs_w, bs_b, bs_w, bs_b, bs_w, bs_b,
                  bs_w, bs_b, bs_hid, bs_bhid, bs_wm2, bs_b,
                  bs_b, bs_b, bs_w, bs_b],
        out_specs=[bs_out, bs_out],
        out_shape=[jax.ShapeDtypeStruct((1, D), jnp.float32),
                   jax.ShapeDtypeStruct((1, D), jnp.float32)],
        scratch_shapes=[pltpu.VMEM((8, D), jnp.float32)],
    )(s2, deg, b2.reshape(1, D),
      Wq, bq.reshape(1, D), Wk, bk.reshape(1, D), Wv, bv.reshape(1, D),
      Wo, bo.reshape(1, D), Wm1, bm1.reshape(1, HID), Wm2, bm2.reshape(1, D),
      g2.reshape(1, D), beta2.reshape(1, D), Wl_p, bl_p)

    return (lg[0, :NC], nr[0])


# speedup vs baseline: 23.3418x; 1.0236x over previous
import functools

import jax
import jax.numpy as jnp
from jax import lax
from jax.experimental import pallas as pl
from jax.experimental.pallas import tpu as pltpu
from jax.experimental.pallas import tpu_sc as plsc

T = 16
NSC = 2
BLK = 128
NPAD = 240
NBUF = 3


def _sc_mesh():
    return plsc.VectorSubcoreMesh(core_axis_name="c", subcore_axis_name="s")


def _deg_body(G, NBLK, RPT,
              dst_hbm, zeros_hbm, ones_hbm, deg_hbm,
              idx_v, ones_v, zero_v, stage_v, sem, acc_sh):
    c = lax.axis_index("c")
    s = lax.axis_index("s")
    pltpu.sync_copy(ones_hbm, ones_v)
    pltpu.sync_copy(zeros_hbm, zero_v)
    for k in range(G // NSC):
        gi = c * (G // NSC) + k
        pltpu.sync_copy(zero_v, acc_sh.at[pl.ds(s * RPT, RPT)])
        plsc.subcore_barrier()
        pltpu.sync_copy(dst_hbm.at[gi, s], idx_v)

        def blk(b):
            pltpu.async_copy(ones_v, acc_sh.at[idx_v.at[b]], sem, add=True)

        def blkw(b):
            pltpu.make_async_copy(ones_v, acc_sh.at[idx_v.at[b]], sem).wait()

        pl.loop(0, NBLK)(blk)
        pl.loop(0, NBLK)(blkw)
        plsc.subcore_barrier()
        pltpu.sync_copy(acc_sh.at[pl.ds(s * RPT, RPT)], stage_v)
        pltpu.sync_copy(stage_v, deg_hbm.at[gi, pl.ds(s * RPT, RPT)])
        plsc.subcore_barrier()


def _make_deg_kernel(G, NP, NBLK):
    RPT = NP // T
    return functools.partial(
        pl.kernel,
        out_type=jax.ShapeDtypeStruct((G, NP, 16), jnp.float32),
        mesh=_sc_mesh(),
        compiler_params=pltpu.CompilerParams(use_tc_tiling_on_sc=False),
        scratch_types=[
            pltpu.VMEM((NBLK, BLK), jnp.int32),
            pltpu.VMEM((BLK, 16), jnp.float32),
            pltpu.VMEM((RPT, 16), jnp.float32),
            pltpu.VMEM((RPT, 16), jnp.float32),
            pltpu.SemaphoreType.DMA,
            pltpu.VMEM_SHARED((NP, 16), jnp.float32),
        ],
    )(functools.partial(_deg_body, G, NBLK, RPT))


def _seg_body(G, NBLK, RPT, IBLK,
              hd_hbm, src_hbm, dst_hbm, out_hbm,
              src_v, dst_v, b0, b1, g0, g1, w0, w1, acc_sh):
    c = lax.axis_index("c")
    s = lax.axis_index("s")
    ring = (b0, b1)
    gsem = (g0, g1)
    ssem = (w0, w1)
    NICH = NBLK // IBLK

    for k in range(G // NSC):
        gi = c * (G // NSC) + k
        r0 = s * RPT
        pltpu.sync_copy(hd_hbm.at[gi, pl.ds(r0, RPT)],
                        acc_sh.at[pl.ds(r0, RPT)])
        plsc.subcore_barrier()

        def gather(j):
            pltpu.async_copy(hd_hbm.at[gi].at[src_v.at[j]], ring[j % 2],
                             gsem[j % 2])

        def gwait(j):
            pltpu.make_async_copy(hd_hbm.at[gi].at[src_v.at[j]], ring[j % 2],
                                  gsem[j % 2]).wait()

        def sstart(j):
            pltpu.async_copy(ring[j % 2], acc_sh.at[dst_v.at[j]],
                             ssem[j % 2], add=True)

        def swait(j):
            pltpu.make_async_copy(ring[j % 2], acc_sh.at[dst_v.at[j]],
                                  ssem[j % 2]).wait()

        def chunk(ic):
            pltpu.sync_copy(src_hbm.at[gi, s, pl.ds(ic * IBLK, IBLK)], src_v)
            pltpu.sync_copy(dst_hbm.at[gi, s, pl.ds(ic * IBLK, IBLK)], dst_v)
            gather(0)
            for j in range(IBLK):
                gwait(j)
                sstart(j)
                if j >= 1:
                    swait(j - 1)
                if j + 1 < IBLK:
                    gather(j + 1)
            swait(IBLK - 1)

        pl.loop(0, NICH)(chunk)
        plsc.subcore_barrier()
        pltpu.sync_copy(acc_sh.at[pl.ds(r0, RPT)],
                        out_hbm.at[gi, pl.ds(r0, RPT)])
        plsc.subcore_barrier()


def _make_seg_kernel(G, NP, D, NBLK, IBLK):
    RPT = NP // T
    return functools.partial(
        pl.kernel,
        out_type=jax.ShapeDtypeStruct((G, NP, D), jnp.float32),
        mesh=_sc_mesh(),
        scratch_types=[
            pltpu.VMEM((IBLK, BLK), jnp.int32),
            pltpu.VMEM((IBLK, BLK), jnp.int32),
            pltpu.VMEM((BLK, D), jnp.float32),
            pltpu.VMEM((BLK, D), jnp.float32),
            pltpu.SemaphoreType.DMA,
            pltpu.SemaphoreType.DMA,
            pltpu.SemaphoreType.DMA,
            pltpu.SemaphoreType.DMA,
            pltpu.VMEM_SHARED((NP, D), jnp.float32),
        ],
    )(functools.partial(_seg_body, G, NBLK, RPT, IBLK))


def _mm1_body(x_ref, deg_ref, w_ref, o_ref):
    dis = lax.rsqrt(deg_ref[0, :, 0:1] + 1.0)
    h = lax.dot_general(x_ref[0], w_ref[...], (((1,), (1,)), ((), ())),
                        preferred_element_type=jnp.float32)
    o_ref[0] = h * dis


def _mm2_body(s_ref, deg_ref, w_ref, b_ref, o_ref):
    dis = lax.rsqrt(deg_ref[0, :, 0:1] + 1.0)
    h = jnp.tanh(s_ref[0] * dis + b_ref[...])
    h = lax.dot_general(h, w_ref[...], (((1,), (1,)), ((), ())),
                        preferred_element_type=jnp.float32)
    o_ref[0] = h * dis


def _c11(a, b):
    return lax.dot_general(a, b, (((1,), (1,)), ((), ())),
                           preferred_element_type=jnp.float32)


def _head_body(G, NT, HEADS, N, R, s_ref, deg_ref, b2_ref,
               wq_ref, bq_ref, wk_ref, bk_ref, wv_ref, bv_ref,
               wo_ref, bo_ref, wm1_ref, bm1_ref, wm2_ref, bm2_ref,
               g2_ref, beta2_ref, wl_ref, bl_ref,
               lg_ref, nr_ref, xacc):
    gi = pl.program_id(0)
    nt = pl.program_id(1)
    dis = lax.rsqrt(deg_ref[0, :, 0:1] + 1.0)
    h = jnp.tanh(s_ref[0] * dis + b2_ref[...])
    rows = lax.broadcasted_iota(jnp.int32, (R, 1), 0) + nt * R
    h = jnp.where(rows < N, h, 0.0)
    part = jnp.sum(h, axis=0, keepdims=True)

    @pl.when(nt == 0)
    def _():
        xacc[pl.ds(gi, 1), :] = part

    @pl.when(nt != 0)
    def _():
        xacc[pl.ds(gi, 1), :] = xacc[pl.ds(gi, 1), :] + part

    @pl.when((gi == G - 1) & (nt == NT - 1))
    def _():
        X = xacc[0:G, :]
        q = _c11(X, wq_ref[...]) + bq_ref[...]
        k_ = _c11(X, wk_ref[...]) + bk_ref[...]
        v = _c11(X, wv_ref[...]) + bv_ref[...]
        dh = X.shape[1] // HEADS
        scale = 1.0 / (float(dh) ** 0.5)
        outs = []
        for hh in range(HEADS):
            sl = slice(hh * dh, (hh + 1) * dh)
            sc = _c11(q[:, sl], k_[:, sl]) * scale
            sc = sc - jnp.max(sc, axis=-1, keepdims=True)
            e = jnp.exp(sc)
            a = e / jnp.sum(e, axis=-1, keepdims=True)
            outs.append(lax.dot_general(a, v[:, sl], (((1,), (0,)), ((), ())),
                                        preferred_element_type=jnp.float32))
        o = jnp.concatenate(outs, axis=1)
        xa = _c11(o, wo_ref[...]) + bo_ref[...]
        m = jnp.maximum(_c11(xa, wm1_ref[...]) + bm1_ref[...], 0.0)
        m = _c11(m, wm2_ref[...]) + bm2_ref[...]
        y = xa + m
        mu = jnp.mean(y, axis=-1, keepdims=True)
        var = jnp.mean((y - mu) ** 2, axis=-1, keepdims=True)
        y = (y - mu) * lax.rsqrt(var + 1e-5) * g2_ref[...] + beta2_ref[...]
        xr = jnp.maximum(y, 0.0)
        nr = jnp.sum(xr, axis=0, keepdims=True)
        nr_ref[...] = nr
        lg_ref[...] = _c11(nr, wl_ref[...]) + bl_ref[...]


def kernel(x, edge_index, W1, b1, W2, b2, Wq, bq, Wk, bk, Wv, bv, Wo, bo,
           Wm1, bm1, Wm2, bm2, g2, beta2, Wl, bl):
    G, N, D = x.shape
    E = edge_index.shape[2]
    HID = Wm1.shape[0]
    NC = Wl.shape[0]
    HEADS = 8
    NP = N + NPAD
    EPT = E // T
    IBLK = 16
    NBLK = -(-EPT // (BLK * IBLK)) * IBLK
    PAD = NBLK * BLK - EPT
    R = 1024
    NT = NP // R

    ei = edge_index.reshape(G, 2, T, EPT)
    ar = jnp.arange(PAD, dtype=jnp.int32)
    toff = jnp.arange(T, dtype=jnp.int32)[None, :, None] * 13
    sent_src = jnp.broadcast_to((ar[None, None, :] * 37 + toff) % N,
                                (G, T, PAD))
    sent_dst = N + (ar[None, None, :] * 7 + toff) % NPAD
    sent_dst = jnp.broadcast_to(sent_dst, (G, T, PAD))
    src_p = jnp.concatenate([ei[:, 0], sent_src], axis=2).reshape(
        G, T, NBLK, BLK)
    dst_p = jnp.concatenate([ei[:, 1], sent_dst], axis=2).reshape(
        G, T, NBLK, BLK)

    zeros_c = jnp.zeros((NP // T, 16), jnp.float32)
    ones_c = jnp.ones((BLK, 16), jnp.float32)

    deg = _make_deg_kernel(G, NP, NBLK)(dst_p, zeros_c, ones_c)

    xp = jnp.pad(x, ((0, 0), (0, NPAD), (0, 0)))
    seg = _make_seg_kernel(G, NP, D, NBLK, IBLK)

    mm_grid = (G, NT)
    bs_row = pl.BlockSpec((1, R, D), lambda g, t: (g, t, 0))
    bs_deg = pl.BlockSpec((1, R, 16), lambda g, t: (g, t, 0))
    bs_w = pl.BlockSpec((D, D), lambda g, t: (0, 0))
    bs_b = pl.BlockSpec((1, D), lambda g, t: (0, 0))
    hd1 = pl.pallas_call(
        _mm1_body,
        grid=mm_grid,
        in_specs=[bs_row, bs_deg, bs_w],
        out_specs=bs_row,
        out_shape=jax.ShapeDtypeStruct((G, NP, D), jnp.float32),
    )(xp, deg, W1)
    s1 = seg(hd1, src_p, dst_p)

    hd2 = pl.pallas_call(
        _mm2_body,
        grid=mm_grid,
        in_specs=[bs_row, bs_deg, bs_w, bs_b],
        out_specs=bs_row,
        out_shape=jax.ShapeDtypeStruct((G, NP, D), jnp.float32),
    )(s1, deg, W2, b1.reshape(1, D))
    s2 = seg(hd2, src_p, dst_p)

    Wl_p = jnp.zeros((D, D), jnp.float32).at[:NC].set(Wl)
    bl_p = jnp.zeros((1, D), jnp.float32).at[0, :NC].set(bl)
    bs_hid = pl.BlockSpec((HID, D), lambda g, t: (0, 0))
    bs_bhid = pl.BlockSpec((1, HID), lambda g, t: (0, 0))
    bs_wm2 = pl.BlockSpec((D, HID), lambda g, t: (0, 0))
    bs_out = pl.BlockSpec((1, D), lambda g, t: (0, 0))
    lg, nr = pl.pallas_call(
        functools.partial(_head_body, G, NT, HEADS, N, R),
        grid=mm_grid,
        in_specs=[bs_row, bs_deg, bs_b,
                  bs_w, bs_b, bs_w, bs_b, bs_w, bs_b,
                  bs_w, bs_b, bs_hid, bs_bhid, bs_wm2, bs_b,
                  bs_b, bs_b, bs_w, bs_b],
        out_specs=[bs_out, bs_out],
        out_shape=[jax.ShapeDtypeStruct((1, D), jnp.float32),
                   jax.ShapeDtypeStruct((1, D), jnp.float32)],
        scratch_shapes=[pltpu.VMEM((8, D), jnp.float32)],
    )(s2, deg, b2.reshape(1, D),
      Wq, bq.reshape(1, D), Wk, bk.reshape(1, D), Wv, bv.reshape(1, D),
      Wo, bo.reshape(1, D), Wm1, bm1.reshape(1, HID), Wm2, bm2.reshape(1, D),
      g2.reshape(1, D), beta2.reshape(1, D), Wl_p, bl_p)

    return (lg[0, :NC], nr[0])


# speedup vs baseline: 25.1016x; 1.0754x over previous
import functools

import jax
import jax.numpy as jnp
from jax import lax
from jax.experimental import pallas as pl
from jax.experimental.pallas import tpu as pltpu
from jax.experimental.pallas import tpu_sc as plsc

T = 16
NSC = 2
BLK = 128
NPAD = 240
IBLK = 16


def _sc_mesh():
    return plsc.VectorSubcoreMesh(core_axis_name="c", subcore_axis_name="s")


def _deg_body(p, NBLK, RPT,
              dst_hbm, zeros_hbm, ones_hbm, deg_hbm,
              idx_v, ones_v, zero_v, stage_v, sem, acc_sh):
    c = lax.axis_index("c")
    s = lax.axis_index("s")
    gi = 2 * c + p
    pltpu.sync_copy(ones_hbm, ones_v)
    pltpu.sync_copy(zeros_hbm, zero_v)
    pltpu.sync_copy(zero_v, acc_sh.at[pl.ds(s * RPT, RPT)])
    plsc.subcore_barrier()
    pltpu.sync_copy(dst_hbm.at[gi, s], idx_v)

    def blk(b):
        pltpu.async_copy(ones_v, acc_sh.at[idx_v.at[b]], sem, add=True)

    def blkw(b):
        pltpu.make_async_copy(ones_v, acc_sh.at[idx_v.at[b]], sem).wait()

    pl.loop(0, NBLK)(blk)
    pl.loop(0, NBLK)(blkw)
    plsc.subcore_barrier()
    pltpu.sync_copy(acc_sh.at[pl.ds(s * RPT, RPT)], stage_v)
    pltpu.sync_copy(stage_v, deg_hbm.at[c, pl.ds(s * RPT, RPT)])


def _make_deg_kernel(p, NP, NBLK):
    RPT = NP // T
    return functools.partial(
        pl.kernel,
        out_type=jax.ShapeDtypeStruct((NSC, NP, 16), jnp.float32),
        mesh=_sc_mesh(),
        compiler_params=pltpu.CompilerParams(use_tc_tiling_on_sc=False),
        scratch_types=[
            pltpu.VMEM((NBLK, BLK), jnp.int32),
            pltpu.VMEM((BLK, 16), jnp.float32),
            pltpu.VMEM((RPT, 16), jnp.float32),
            pltpu.VMEM((RPT, 16), jnp.float32),
            pltpu.SemaphoreType.DMA,
            pltpu.VMEM_SHARED((NP, 16), jnp.float32),
        ],
    )(functools.partial(_deg_body, p, NBLK, RPT))


def _seg_body(p, NBLK, RPT,
              hd_hbm, src_hbm, dst_hbm, out_hbm,
              src_v, dst_v, b0, b1, g0, g1, w0, w1, acc_sh):
    c = lax.axis_index("c")
    s = lax.axis_index("s")
    gi = 2 * c + p
    ring = (b0, b1)
    gsem = (g0, g1)
    ssem = (w0, w1)
    NICH = NBLK // IBLK

    r0 = s * RPT
    pltpu.sync_copy(hd_hbm.at[c, pl.ds(r0, RPT)], acc_sh.at[pl.ds(r0, RPT)])
    plsc.subcore_barrier()

    def gather(j):
        pltpu.async_copy(hd_hbm.at[c].at[src_v.at[j]], ring[j % 2],
                         gsem[j % 2])

    def gwait(j):
        pltpu.make_async_copy(hd_hbm.at[c].at[src_v.at[j]], ring[j % 2],
                              gsem[j % 2]).wait()

    def sstart(j):
        pltpu.async_copy(ring[j % 2], acc_sh.at[dst_v.at[j]],
                         ssem[j % 2], add=True)

    def swait(j):
        pltpu.make_async_copy(ring[j % 2], acc_sh.at[dst_v.at[j]],
                              ssem[j % 2]).wait()

    def chunk(ic):
        pltpu.sync_copy(src_hbm.at[gi, s, pl.ds(ic * IBLK, IBLK)], src_v)
        pltpu.sync_copy(dst_hbm.at[gi, s, pl.ds(ic * IBLK, IBLK)], dst_v)
        gather(0)
        for j in range(IBLK):
            gwait(j)
            sstart(j)
            if j >= 1:
                swait(j - 1)
            if j + 1 < IBLK:
                gather(j + 1)
        swait(IBLK - 1)

    pl.loop(0, NICH)(chunk)
    plsc.subcore_barrier()
    pltpu.sync_copy(acc_sh.at[pl.ds(r0, RPT)], out_hbm.at[c, pl.ds(r0, RPT)])


def _make_seg_kernel(p, NP, D, NBLK):
    RPT = NP // T
    return functools.partial(
        pl.kernel,
        out_type=jax.ShapeDtypeStruct((NSC, NP, D), jnp.float32),
        mesh=_sc_mesh(),
        scratch_types=[
            pltpu.VMEM((IBLK, BLK), jnp.int32),
            pltpu.VMEM((IBLK, BLK), jnp.int32),
            pltpu.VMEM((BLK, D), jnp.float32),
            pltpu.VMEM((BLK, D), jnp.float32),
            pltpu.SemaphoreType.DMA,
            pltpu.SemaphoreType.DMA,
            pltpu.SemaphoreType.DMA,
            pltpu.SemaphoreType.DMA,
            pltpu.VMEM_SHARED((NP, D), jnp.float32),
        ],
    )(functools.partial(_seg_body, p, NBLK, RPT))


def _mm1_body(x_ref, deg_ref, w_ref, o_ref):
    dis = lax.rsqrt(deg_ref[0, :, 0:1] + 1.0)
    h = lax.dot_general(x_ref[0], w_ref[...], (((1,), (1,)), ((), ())),
                        preferred_element_type=jnp.float32)
    o_ref[0] = h * dis


def _mm2_body(s_ref, deg_ref, w_ref, b_ref, o_ref):
    dis = lax.rsqrt(deg_ref[0, :, 0:1] + 1.0)
    h = jnp.tanh(s_ref[0] * dis + b_ref[...])
    h = lax.dot_general(h, w_ref[...], (((1,), (1,)), ((), ())),
                        preferred_element_type=jnp.float32)
    o_ref[0] = h * dis


def _reduce_body(NT, N, R, s_ref, deg_ref, b2_ref, x_ref, xacc):
    gi = pl.program_id(0)
    nt = pl.program_id(1)
    dis = lax.rsqrt(deg_ref[0, :, 0:1] + 1.0)
    h = jnp.tanh(s_ref[0] * dis + b2_ref[...])
    rows = lax.broadcasted_iota(jnp.int32, (R, 1), 0) + nt * R
    h = jnp.where(rows < N, h, 0.0)
    part = jnp.sum(h, axis=0, keepdims=True)

    @pl.when(nt == 0)
    def _():
        xacc[...] = part

    @pl.when(nt != 0)
    def _():
        xacc[...] = xacc[...] + part

    @pl.when(nt == NT - 1)
    def _():
        x_ref[pl.ds(gi, 1), :] = xacc[...]


def _c11(a, b):
    return lax.dot_general(a, b, (((1,), (1,)), ((), ())),
                           preferred_element_type=jnp.float32)


def _head_body(G, HEADS, xa_ref, xb_ref,
               wq_ref, bq_ref, wk_ref, bk_ref, wv_ref, bv_ref,
               wo_ref, bo_ref, wm1_ref, bm1_ref, wm2_ref, bm2_ref,
               g2_ref, beta2_ref, wl_ref, bl_ref,
               lg_ref, nr_ref):
    X = jnp.concatenate([xa_ref[0:1], xb_ref[0:1],
                         xa_ref[1:2], xb_ref[1:2]], axis=0)
    q = _c11(X, wq_ref[...]) + bq_ref[...]
    k_ = _c11(X, wk_ref[...]) + bk_ref[...]
    v = _c11(X, wv_ref[...]) + bv_ref[...]
    dh = X.shape[1] // HEADS
    scale = 1.0 / (float(dh) ** 0.5)
    outs = []
    for hh in range(HEADS):
        sl = slice(hh * dh, (hh + 1) * dh)
        sc = _c11(q[:, sl], k_[:, sl]) * scale
        sc = sc - jnp.max(sc, axis=-1, keepdims=True)
        e = jnp.exp(sc)
        a = e / jnp.sum(e, axis=-1, keepdims=True)
        outs.append(lax.dot_general(a, v[:, sl], (((1,), (0,)), ((), ())),
                                    preferred_element_type=jnp.float32))
    o = jnp.concatenate(outs, axis=1)
    xa = _c11(o, wo_ref[...]) + bo_ref[...]
    m = jnp.maximum(_c11(xa, wm1_ref[...]) + bm1_ref[...], 0.0)
    m = _c11(m, wm2_ref[...]) + bm2_ref[...]
    y = xa + m
    mu = jnp.mean(y, axis=-1, keepdims=True)
    var = jnp.mean((y - mu) ** 2, axis=-1, keepdims=True)
    y = (y - mu) * lax.rsqrt(var + 1e-5) * g2_ref[...] + beta2_ref[...]
    xr = jnp.maximum(y, 0.0)
    nr = jnp.sum(xr, axis=0, keepdims=True)
    nr_ref[...] = nr
    lg_ref[...] = _c11(nr, wl_ref[...]) + bl_ref[...]


def kernel(x, edge_index, W1, b1, W2, b2, Wq, bq, Wk, bk, Wv, bv, Wo, bo,
           Wm1, bm1, Wm2, bm2, g2, beta2, Wl, bl):
    G, N, D = x.shape
    E = edge_index.shape[2]
    HID = Wm1.shape[0]
    NC = Wl.shape[0]
    HEADS = 8
    NP = N + NPAD
    EPT = E // T
    NBLK = -(-EPT // (BLK * IBLK)) * IBLK
    PAD = NBLK * BLK - EPT
    R = 1024
    NT = NP // R

    ei = edge_index.reshape(G, 2, T, EPT)
    ar = jnp.arange(PAD, dtype=jnp.int32)
    toff = jnp.arange(T, dtype=jnp.int32)[None, :, None] * 13
    sent_src = jnp.broadcast_to((ar[None, None, :] * 37 + toff) % N,
                                (G, T, PAD))
    sent_dst = N + (ar[None, None, :] * 7 + toff) % NPAD
    sent_dst = jnp.broadcast_to(sent_dst, (G, T, PAD))
    src_p = jnp.concatenate([ei[:, 0], sent_src], axis=2).reshape(
        G, T, NBLK, BLK)
    dst_p = jnp.concatenate([ei[:, 1], sent_dst], axis=2).reshape(
        G, T, NBLK, BLK)

    zeros_c = jnp.zeros((NP // T, 16), jnp.float32)
    ones_c = jnp.ones((BLK, 16), jnp.float32)

    xp = jnp.pad(x, ((0, 0), (0, NPAD), (0, 0)))

    mm_grid = (NSC, NT)
    bs_deg = pl.BlockSpec((1, R, 16), lambda g, t: (g, t, 0))
    bs_w = pl.BlockSpec((D, D), lambda g, t: (0, 0))
    bs_b = pl.BlockSpec((1, D), lambda g, t: (0, 0))
    bs_row = pl.BlockSpec((1, R, D), lambda g, t: (g, t, 0))

    def bs_xrow(p):
        return pl.BlockSpec((1, R, D), lambda g, t: (2 * g + p, t, 0))

    Xs = []
    for p in range(NSC):
        deg = _make_deg_kernel(p, NP, NBLK)(dst_p, zeros_c, ones_c)
        hd1 = pl.pallas_call(
            _mm1_body,
            grid=mm_grid,
            in_specs=[bs_xrow(p), bs_deg, bs_w],
            out_specs=bs_row,
            out_shape=jax.ShapeDtypeStruct((NSC, NP, D), jnp.float32),
        )(xp, deg, W1)
        s1 = _make_seg_kernel(p, NP, D, NBLK)(hd1, src_p, dst_p)
        hd2 = pl.pallas_call(
            _mm2_body,
            grid=mm_grid,
            in_specs=[bs_row, bs_deg, bs_w, bs_b],
            out_specs=bs_row,
            out_shape=jax.ShapeDtypeStruct((NSC, NP, D), jnp.float32),
        )(s1, deg, W2, b1.reshape(1, D))
        s2 = _make_seg_kernel(p, NP, D, NBLK)(hd2, src_p, dst_p)
        Xp = pl.pallas_call(
            functools.partial(_reduce_body, NT, N, R),
            grid=mm_grid,
            in_specs=[bs_row, bs_deg, bs_b],
            out_specs=pl.BlockSpec((NSC, D), lambda g, t: (0, 0)),
            out_shape=jax.ShapeDtypeStruct((NSC, D), jnp.float32),
            scratch_shapes=[pltpu.VMEM((1, D), jnp.float32)],
        )(s2, deg, b2.reshape(1, D))
        Xs.append(Xp)

    Wl_p = jnp.zeros((D, D), jnp.float32).at[:NC].set(Wl)
    bl_p = jnp.zeros((1, D), jnp.float32).at[0, :NC].set(bl)
    bs_x = pl.BlockSpec((NSC, D), lambda: (0, 0))
    bs_w0 = pl.BlockSpec((D, D), lambda: (0, 0))
    bs_b0 = pl.BlockSpec((1, D), lambda: (0, 0))
    lg, nr = pl.pallas_call(
        functools.partial(_head_body, G, HEADS),
        grid=(),
        in_specs=[bs_x, bs_x,
                  bs_w0, bs_b0, bs_w0, bs_b0, bs_w0, bs_b0,
                  bs_w0, bs_b0, pl.BlockSpec((HID, D), lambda: (0, 0)),
                  pl.BlockSpec((1, HID), lambda: (0, 0)),
                  pl.BlockSpec((D, HID), lambda: (0, 0)), bs_b0,
                  bs_b0, bs_b0, bs_w0, bs_b0],
        out_specs=[bs_b0, bs_b0],
        out_shape=[jax.ShapeDtypeStruct((1, D), jnp.float32),
                   jax.ShapeDtypeStruct((1, D), jnp.float32)],
    )(Xs[0], Xs[1],
      Wq, bq.reshape(1, D), Wk, bk.reshape(1, D), Wv, bv.reshape(1, D),
      Wo, bo.reshape(1, D), Wm1, bm1.reshape(1, HID), Wm2, bm2.reshape(1, D),
      g2.reshape(1, D), beta2.reshape(1, D), Wl_p, bl_p)

    return (lg[0, :NC], nr[0])


# speedup vs baseline: 25.3460x; 1.0097x over previous
import functools

import jax
import jax.numpy as jnp
from jax import lax
from jax.experimental import pallas as pl
from jax.experimental.pallas import tpu as pltpu
from jax.experimental.pallas import tpu_sc as plsc

T = 16
NSC = 2
BLK = 128
NPAD = 240
IBLK = 32


def _sc_mesh():
    return plsc.VectorSubcoreMesh(core_axis_name="c", subcore_axis_name="s")


def _deg_body(p, NBLK, RPT,
              dst_hbm, zeros_hbm, ones_hbm, deg_hbm,
              idx_v, ones_v, zero_v, stage_v, sem, acc_sh):
    c = lax.axis_index("c")
    s = lax.axis_index("s")
    gi = 2 * c + p
    pltpu.sync_copy(ones_hbm, ones_v)
    pltpu.sync_copy(zeros_hbm, zero_v)
    pltpu.sync_copy(zero_v, acc_sh.at[pl.ds(s * RPT, RPT)])
    plsc.subcore_barrier()
    pltpu.sync_copy(dst_hbm.at[gi, s], idx_v)

    def blk(b):
        pltpu.async_copy(ones_v, acc_sh.at[idx_v.at[b]], sem, add=True)

    def blkw(b):
        pltpu.make_async_copy(ones_v, acc_sh.at[idx_v.at[b]], sem).wait()

    pl.loop(0, NBLK)(blk)
    pl.loop(0, NBLK)(blkw)
    plsc.subcore_barrier()
    pltpu.sync_copy(acc_sh.at[pl.ds(s * RPT, RPT)], stage_v)
    pltpu.sync_copy(stage_v, deg_hbm.at[c, pl.ds(s * RPT, RPT)])


def _make_deg_kernel(p, NP, NBLK):
    RPT = NP // T
    return functools.partial(
        pl.kernel,
        out_type=jax.ShapeDtypeStruct((NSC, NP, 16), jnp.float32),
        mesh=_sc_mesh(),
        compiler_params=pltpu.CompilerParams(use_tc_tiling_on_sc=False),
        scratch_types=[
            pltpu.VMEM((NBLK, BLK), jnp.int32),
            pltpu.VMEM((BLK, 16), jnp.float32),
            pltpu.VMEM((RPT, 16), jnp.float32),
            pltpu.VMEM((RPT, 16), jnp.float32),
            pltpu.SemaphoreType.DMA,
            pltpu.VMEM_SHARED((NP, 16), jnp.float32),
        ],
    )(functools.partial(_deg_body, p, NBLK, RPT))


def _seg_body(p, NBLK, RPT,
              hd_hbm, src_hbm, dst_hbm, out_hbm,
              src_v, dst_v, b0, b1, g0, g1, w0, w1, acc_sh):
    c = lax.axis_index("c")
    s = lax.axis_index("s")
    gi = 2 * c + p
    ring = (b0, b1)
    gsem = (g0, g1)
    ssem = (w0, w1)
    NICH = NBLK // IBLK

    r0 = s * RPT
    pltpu.sync_copy(hd_hbm.at[c, pl.ds(r0, RPT)], acc_sh.at[pl.ds(r0, RPT)])
    plsc.subcore_barrier()

    def gather(j):
        pltpu.async_copy(hd_hbm.at[c].at[src_v.at[j]], ring[j % 2],
                         gsem[j % 2])

    def gwait(j):
        pltpu.make_async_copy(hd_hbm.at[c].at[src_v.at[j]], ring[j % 2],
                              gsem[j % 2]).wait()

    def sstart(j):
        pltpu.async_copy(ring[j % 2], acc_sh.at[dst_v.at[j]],
                         ssem[j % 2], add=True)

    def swait(j):
        pltpu.make_async_copy(ring[j % 2], acc_sh.at[dst_v.at[j]],
                              ssem[j % 2]).wait()

    def chunk(ic):
        pltpu.sync_copy(src_hbm.at[gi, s, pl.ds(ic * IBLK, IBLK)], src_v)
        pltpu.sync_copy(dst_hbm.at[gi, s, pl.ds(ic * IBLK, IBLK)], dst_v)
        gather(0)
        for j in range(IBLK):
            gwait(j)
            sstart(j)
            if j >= 1:
                swait(j - 1)
            if j + 1 < IBLK:
                gather(j + 1)
        swait(IBLK - 1)

    pl.loop(0, NICH)(chunk)
    plsc.subcore_barrier()
    pltpu.sync_copy(acc_sh.at[pl.ds(r0, RPT)], out_hbm.at[c, pl.ds(r0, RPT)])


def _make_seg_kernel(p, NP, D, NBLK):
    RPT = NP // T
    return functools.partial(
        pl.kernel,
        out_type=jax.ShapeDtypeStruct((NSC, NP, D), jnp.float32),
        mesh=_sc_mesh(),
        scratch_types=[
            pltpu.VMEM((IBLK, BLK), jnp.int32),
            pltpu.VMEM((IBLK, BLK), jnp.int32),
            pltpu.VMEM((BLK, D), jnp.float32),
            pltpu.VMEM((BLK, D), jnp.float32),
            pltpu.SemaphoreType.DMA,
            pltpu.SemaphoreType.DMA,
            pltpu.SemaphoreType.DMA,
            pltpu.SemaphoreType.DMA,
            pltpu.VMEM_SHARED((NP, D), jnp.float32),
        ],
    )(functools.partial(_seg_body, p, NBLK, RPT))


def _mm1_body(x_ref, deg_ref, w_ref, o_ref):
    dis = lax.rsqrt(deg_ref[0, :, 0:1] + 1.0)
    h = lax.dot_general(x_ref[0], w_ref[...], (((1,), (1,)), ((), ())),
                        preferred_element_type=jnp.float32)
    o_ref[0] = h * dis


def _mm2_body(s_ref, deg_ref, w_ref, b_ref, o_ref):
    dis = lax.rsqrt(deg_ref[0, :, 0:1] + 1.0)
    h = jnp.tanh(s_ref[0] * dis + b_ref[...])
    h = lax.dot_general(h, w_ref[...], (((1,), (1,)), ((), ())),
                        preferred_element_type=jnp.float32)
    o_ref[0] = h * dis


def _reduce_body(NT, N, R, s_ref, deg_ref, b2_ref, x_ref, xacc):
    gi = pl.program_id(0)
    nt = pl.program_id(1)
    dis = lax.rsqrt(deg_ref[0, :, 0:1] + 1.0)
    h = jnp.tanh(s_ref[0] * dis + b2_ref[...])
    rows = lax.broadcasted_iota(jnp.int32, (R, 1), 0) + nt * R
    h = jnp.where(rows < N, h, 0.0)
    part = jnp.sum(h, axis=0, keepdims=True)

    @pl.when(nt == 0)
    def _():
        xacc[...] = part

    @pl.when(nt != 0)
    def _():
        xacc[...] = xacc[...] + part

    @pl.when(nt == NT - 1)
    def _():
        x_ref[pl.ds(gi, 1), :] = xacc[...]


def _c11(a, b):
    return lax.dot_general(a, b, (((1,), (1,)), ((), ())),
                           preferred_element_type=jnp.float32)


def _head_body(G, HEADS, xa_ref, xb_ref,
               wq_ref, bq_ref, wk_ref, bk_ref, wv_ref, bv_ref,
               wo_ref, bo_ref, wm1_ref, bm1_ref, wm2_ref, bm2_ref,
               g2_ref, beta2_ref, wl_ref, bl_ref,
               lg_ref, nr_ref):
    X = jnp.concatenate([xa_ref[0:1], xb_ref[0:1],
                         xa_ref[1:2], xb_ref[1:2]], axis=0)
    q = _c11(X, wq_ref[...]) + bq_ref[...]
    k_ = _c11(X, wk_ref[...]) + bk_ref[...]
    v = _c11(X, wv_ref[...]) + bv_ref[...]
    dh = X.shape[1] // HEADS
    scale = 1.0 / (float(dh) ** 0.5)
    outs = []
    for hh in range(HEADS):
        sl = slice(hh * dh, (hh + 1) * dh)
        sc = _c11(q[:, sl], k_[:, sl]) * scale
        sc = sc - jnp.max(sc, axis=-1, keepdims=True)
        e = jnp.exp(sc)
        a = e / jnp.sum(e, axis=-1, keepdims=True)
        outs.append(lax.dot_general(a, v[:, sl], (((1,), (0,)), ((), ())),
                                    preferred_element_type=jnp.float32))
    o = jnp.concatenate(outs, axis=1)
    xa = _c11(o, wo_ref[...]) + bo_ref[...]
    m = jnp.maximum(_c11(xa, wm1_ref[...]) + bm1_ref[...], 0.0)
    m = _c11(m, wm2_ref[...]) + bm2_ref[...]
    y = xa + m
    mu = jnp.mean(y, axis=-1, keepdims=True)
    var = jnp.mean((y - mu) ** 2, axis=-1, keepdims=True)
    y = (y - mu) * lax.rsqrt(var + 1e-5) * g2_ref[...] + beta2_ref[...]
    xr = jnp.maximum(y, 0.0)
    nr = jnp.sum(xr, axis=0, keepdims=True)
    nr_ref[...] = nr
    lg_ref[...] = _c11(nr, wl_ref[...]) + bl_ref[...]


def kernel(x, edge_index, W1, b1, W2, b2, Wq, bq, Wk, bk, Wv, bv, Wo, bo,
           Wm1, bm1, Wm2, bm2, g2, beta2, Wl, bl):
    G, N, D = x.shape
    E = edge_index.shape[2]
    HID = Wm1.shape[0]
    NC = Wl.shape[0]
    HEADS = 8
    NP = N + NPAD
    EPT = E // T
    NBLK = -(-EPT // (BLK * IBLK)) * IBLK
    PAD = NBLK * BLK - EPT
    R = 1024
    NT = NP // R

    ei = edge_index.reshape(G, 2, T, EPT)
    ar = jnp.arange(PAD, dtype=jnp.int32)
    toff = jnp.arange(T, dtype=jnp.int32)[None, :, None] * 13
    sent_src = jnp.broadcast_to((ar[None, None, :] * 37 + toff) % N,
                                (G, T, PAD))
    sent_dst = N + (ar[None, None, :] * 7 + toff) % NPAD
    sent_dst = jnp.broadcast_to(sent_dst, (G, T, PAD))
    src_p = jnp.concatenate([ei[:, 0], sent_src], axis=2).reshape(
        G, T, NBLK, BLK)
    dst_p = jnp.concatenate([ei[:, 1], sent_dst], axis=2).reshape(
        G, T, NBLK, BLK)

    zeros_c = jnp.zeros((NP // T, 16), jnp.float32)
    ones_c = jnp.ones((BLK, 16), jnp.float32)

    xp = jnp.pad(x, ((0, 0), (0, NPAD), (0, 0)))

    mm_grid = (NSC, NT)
    bs_deg = pl.BlockSpec((1, R, 16), lambda g, t: (g, t, 0))
    bs_w = pl.BlockSpec((D, D), lambda g, t: (0, 0))
    bs_b = pl.BlockSpec((1, D), lambda g, t: (0, 0))
    bs_row = pl.BlockSpec((1, R, D), lambda g, t: (g, t, 0))

    def bs_xrow(p):
        return pl.BlockSpec((1, R, D), lambda g, t: (2 * g + p, t, 0))

    Xs = []
    for p in range(NSC):
        deg = _make_deg_kernel(p, NP, NBLK)(dst_p, zeros_c, ones_c)
        hd1 = pl.pallas_call(
            _mm1_body,
            grid=mm_grid,
            in_specs=[bs_xrow(p), bs_deg, bs_w],
            out_specs=bs_row,
            out_shape=jax.ShapeDtypeStruct((NSC, NP, D), jnp.float32),
        )(xp, deg, W1)
        s1 = _make_seg_kernel(p, NP, D, NBLK)(hd1, src_p, dst_p)
        hd2 = pl.pallas_call(
            _mm2_body,
            grid=mm_grid,
            in_specs=[bs_row, bs_deg, bs_w, bs_b],
            out_specs=bs_row,
            out_shape=jax.ShapeDtypeStruct((NSC, NP, D), jnp.float32),
        )(s1, deg, W2, b1.reshape(1, D))
        s2 = _make_seg_kernel(p, NP, D, NBLK)(hd2, src_p, dst_p)
        Xp = pl.pallas_call(
            functools.partial(_reduce_body, NT, N, R),
            grid=mm_grid,
            in_specs=[bs_row, bs_deg, bs_b],
            out_specs=pl.BlockSpec((NSC, D), lambda g, t: (0, 0)),
            out_shape=jax.ShapeDtypeStruct((NSC, D), jnp.float32),
            scratch_shapes=[pltpu.VMEM((1, D), jnp.float32)],
        )(s2, deg, b2.reshape(1, D))
        Xs.append(Xp)

    Wl_p = jnp.zeros((D, D), jnp.float32).at[:NC].set(Wl)
    bl_p = jnp.zeros((1, D), jnp.float32).at[0, :NC].set(bl)
    bs_x = pl.BlockSpec((NSC, D), lambda: (0, 0))
    bs_w0 = pl.BlockSpec((D, D), lambda: (0, 0))
    bs_b0 = pl.BlockSpec((1, D), lambda: (0, 0))
    lg, nr = pl.pallas_call(
        functools.partial(_head_body, G, HEADS),
        grid=(),
        in_specs=[bs_x, bs_x,
                  bs_w0, bs_b0, bs_w0, bs_b0, bs_w0, bs_b0,
                  bs_w0, bs_b0, pl.BlockSpec((HID, D), lambda: (0, 0)),
                  pl.BlockSpec((1, HID), lambda: (0, 0)),
                  pl.BlockSpec((D, HID), lambda: (0, 0)), bs_b0,
                  bs_b0, bs_b0, bs_w0, bs_b0],
        out_specs=[bs_b0, bs_b0],
        out_shape=[jax.ShapeDtypeStruct((1, D), jnp.float32),
                   jax.ShapeDtypeStruct((1, D), jnp.float32)],
    )(Xs[0], Xs[1],
      Wq, bq.reshape(1, D), Wk, bk.reshape(1, D), Wv, bv.reshape(1, D),
      Wo, bo.reshape(1, D), Wm1, bm1.reshape(1, HID), Wm2, bm2.reshape(1, D),
      g2.reshape(1, D), beta2.reshape(1, D), Wl_p, bl_p)

    return (lg[0, :NC], nr[0])
